# Initial kernel scaffold; baseline (speedup 1.0000x reference)
import jax
import jax.numpy as jnp
from jax.experimental import pallas as pl


def kernel(queries, keys, k):
    raise NotImplementedError("write your pallas kernel here")


# speedup vs baseline: 2.6512x; 2.6512x over previous
import functools

import jax
import jax.numpy as jnp
from jax import lax
from jax.experimental import pallas as pl
from jax.experimental.pallas import tpu as pltpu

NQ = 4096
NK = 100000
ND = 128
KNN = 32
KPAD = 102400
SEGW = 32
NSEG = KPAD // SEGW
QB = 256
KB = 4096
NCAND = KNN * SEGW
BIG = 1 << 30
NEG_INF = float("-inf")


def _scores_body(q_ref, k_ref, ksq_ref, s_ref, m_ref):
    q = q_ref[...]
    kk = k_ref[...]
    dots = lax.dot_general(q, kk, (((1,), (1,)), ((), ())),
                           preferred_element_type=jnp.float32)
    qsq = jnp.sum(q * q, axis=1, keepdims=True)
    neg = -(qsq - 2.0 * dots + ksq_ref[...])
    s_ref[...] = neg
    m_ref[...] = jnp.max(neg.reshape(QB, KB // SEGW, SEGW), axis=2)


def _scores(queries, keys_p, ksq):
    grid = (NQ // QB, KPAD // KB)
    return pl.pallas_call(
        _scores_body,
        grid=grid,
        in_specs=[
            pl.BlockSpec((QB, ND), lambda qi, ki: (qi, 0)),
            pl.BlockSpec((KB, ND), lambda qi, ki: (ki, 0)),
            pl.BlockSpec((1, KB), lambda qi, ki: (0, ki)),
        ],
        out_specs=[
            pl.BlockSpec((QB, KB), lambda qi, ki: (qi, ki)),
            pl.BlockSpec((QB, KB // SEGW), lambda qi, ki: (qi, ki)),
        ],
        out_shape=[
            jax.ShapeDtypeStruct((NQ, KPAD), jnp.float32),
            jax.ShapeDtypeStruct((NQ, NSEG), jnp.float32),
        ],
    )(queries, keys_p, ksq)


def _segtop_body(m_ref, seg_ref, flat_ref):
    x = m_ref[...]
    qi = pl.program_id(0)
    lane = lax.broadcasted_iota(jnp.int32, (QB, NSEG), 1)
    col = lax.broadcasted_iota(jnp.int32, (QB, KNN), 1)
    seg = jnp.zeros((QB, KNN), jnp.int32)
    for it in range(KNN):
        m = jnp.max(x, axis=1, keepdims=True)
        p = jnp.min(jnp.where(x == m, lane, BIG), axis=1)
        seg = jnp.where(col == it, p[:, None], seg)
        x = jnp.where(lane == p[:, None], NEG_INF, x)
    rows = lax.broadcasted_iota(jnp.int32, (QB, KNN), 0) + qi * QB
    seg_ref[...] = seg
    flat_ref[...] = rows * NSEG + seg


def _segtop(m):
    return pl.pallas_call(
        _segtop_body,
        grid=(NQ // QB,),
        in_specs=[pl.BlockSpec((QB, NSEG), lambda qi: (qi, 0))],
        out_specs=[
            pl.BlockSpec((QB, KNN), lambda qi: (qi, 0)),
            pl.BlockSpec((QB, KNN), lambda qi: (qi, 0)),
        ],
        out_shape=[
            jax.ShapeDtypeStruct((NQ, KNN), jnp.int32),
            jax.ShapeDtypeStruct((NQ, KNN), jnp.int32),
        ],
    )(m)


def _final_body(c_ref, seg_ref, val_ref, idx_ref):
    x = c_ref[...]
    seg = seg_ref[...]
    off = lax.broadcasted_iota(jnp.int32, (QB, KNN, SEGW), 2)
    gidx = (seg[:, :, None] * SEGW + off).reshape(QB, NCAND)
    lane = lax.broadcasted_iota(jnp.int32, (QB, NCAND), 1)
    col = lax.broadcasted_iota(jnp.int32, (QB, KNN), 1)
    accv = jnp.zeros((QB, KNN), jnp.float32)
    acci = jnp.zeros((QB, KNN), jnp.int32)
    for it in range(KNN):
        m = jnp.max(x, axis=1, keepdims=True)
        p = jnp.min(jnp.where(x == m, lane, BIG), axis=1)[:, None]
        gi = jnp.max(jnp.where(lane == p, gidx, -1), axis=1)[:, None]
        accv = jnp.where(col == it, m, accv)
        acci = jnp.where(col == it, gi, acci)
        x = jnp.where(lane == p, NEG_INF, x)
    val_ref[...] = -accv
    idx_ref[...] = acci


def _final(cand, seg_ids):
    return pl.pallas_call(
        _final_body,
        grid=(NQ // QB,),
        in_specs=[
            pl.BlockSpec((QB, NCAND), lambda qi: (qi, 0)),
            pl.BlockSpec((QB, KNN), lambda qi: (qi, 0)),
        ],
        out_specs=[
            pl.BlockSpec((QB, KNN), lambda qi: (qi, 0)),
            pl.BlockSpec((QB, KNN), lambda qi: (qi, 0)),
        ],
        out_shape=[
            jax.ShapeDtypeStruct((NQ, KNN), jnp.float32),
            jax.ShapeDtypeStruct((NQ, KNN), jnp.int32),
        ],
    )(cand, seg_ids)


def kernel(queries, keys, k):
    del k
    keys_p = jnp.pad(keys, ((0, KPAD - NK), (0, 0)), constant_values=1000.0)
    ksq = jnp.sum(keys_p * keys_p, axis=1)
    s, m = _scores(queries, keys_p, ksq.reshape(1, KPAD))
    seg_ids, flat_idx = _segtop(m)
    s_rows = s.reshape(NQ * NSEG, SEGW)
    cand = s_rows[flat_idx.reshape(-1)].reshape(NQ, NCAND)
    vals, idxs = _final(cand, seg_ids)
    return vals, idxs


# speedup vs baseline: 6.5314x; 2.4636x over previous
import functools

import jax
import jax.numpy as jnp
from jax import lax
from jax.experimental import pallas as pl
from jax.experimental.pallas import tpu as pltpu
from jax.experimental.pallas import tpu_sc as plsc

NQ = 4096
NK = 100000
ND = 128
KNN = 32
KPAD = 102400
SEGW = 32
NSEG = KPAD // SEGW
QB = 256
KB = 4096
NCAND = KNN * SEGW
GROW = 128
NROW = KPAD // GROW
BIG = 1 << 30
NEG_INF = float("-inf")


def _scores_body(q_ref, k_ref, ksq_ref, s_ref, m_ref):
    q = q_ref[...]
    kk = k_ref[...]
    dots = lax.dot_general(q, kk, (((1,), (1,)), ((), ())),
                           preferred_element_type=jnp.float32)
    qsq = jnp.sum(q * q, axis=1, keepdims=True)
    neg = -(qsq - 2.0 * dots + ksq_ref[...])
    s_ref[...] = neg.reshape(QB, KB // GROW, GROW)
    m_ref[...] = jnp.max(neg.reshape(QB, KB // SEGW, SEGW), axis=2)


def _scores(queries, keys_p, ksq):
    grid = (NQ // QB, KPAD // KB)
    return pl.pallas_call(
        _scores_body,
        grid=grid,
        in_specs=[
            pl.BlockSpec((QB, ND), lambda qi, ki: (qi, 0)),
            pl.BlockSpec((KB, ND), lambda qi, ki: (ki, 0)),
            pl.BlockSpec((1, KB), lambda qi, ki: (0, ki)),
        ],
        out_specs=[
            pl.BlockSpec((QB, KB // GROW, GROW), lambda qi, ki: (qi, ki, 0)),
            pl.BlockSpec((QB, KB // SEGW), lambda qi, ki: (qi, ki)),
        ],
        out_shape=[
            jax.ShapeDtypeStruct((NQ, NROW, GROW), jnp.float32),
            jax.ShapeDtypeStruct((NQ, NSEG), jnp.float32),
        ],
    )(queries, keys_p, ksq)


def _segtop_body(m_ref, seg_ref, flat_ref):
    x = m_ref[...]
    qi = pl.program_id(0)
    lane = lax.broadcasted_iota(jnp.int32, (QB, NSEG), 1)
    col = lax.broadcasted_iota(jnp.int32, (QB, KNN), 1)
    seg = jnp.zeros((QB, KNN), jnp.int32)
    for it in range(KNN):
        m = jnp.max(x, axis=1, keepdims=True)
        p = jnp.min(jnp.where(x == m, lane, BIG), axis=1)
        seg = jnp.where(col == it, p[:, None], seg)
        x = jnp.where(lane == p[:, None], NEG_INF, x)
    rows = lax.broadcasted_iota(jnp.int32, (QB, KNN), 0) + qi * QB
    seg_ref[...] = seg
    flat_ref[...] = rows * NROW + (seg >> 2)


def _segtop(m):
    return pl.pallas_call(
        _segtop_body,
        grid=(NQ // QB,),
        in_specs=[pl.BlockSpec((QB, NSEG), lambda qi: (qi, 0))],
        out_specs=[
            pl.BlockSpec((QB, KNN), lambda qi: (qi, 0)),
            pl.BlockSpec((QB, KNN), lambda qi: (qi, 0)),
        ],
        out_shape=[
            jax.ShapeDtypeStruct((NQ, KNN), jnp.int32),
            jax.ShapeDtypeStruct((NQ, KNN), jnp.int32),
        ],
    )(m)


def _final_body(c_ref, seg_ref, val_ref, idx_ref):
    c = c_ref[...]
    seg = seg_ref[...]
    w = (seg & 3)[:, :, None]
    x = jnp.where(w == 0, c[:, :, 0:SEGW],
        jnp.where(w == 1, c[:, :, SEGW:2 * SEGW],
        jnp.where(w == 2, c[:, :, 2 * SEGW:3 * SEGW], c[:, :, 3 * SEGW:])))
    x = x.reshape(QB, NCAND)
    off = lax.broadcasted_iota(jnp.int32, (QB, KNN, SEGW), 2)
    gidx = (seg[:, :, None] * SEGW + off).reshape(QB, NCAND)
    lane = lax.broadcasted_iota(jnp.int32, (QB, NCAND), 1)
    col = lax.broadcasted_iota(jnp.int32, (QB, KNN), 1)
    accv = jnp.zeros((QB, KNN), jnp.float32)
    acci = jnp.zeros((QB, KNN), jnp.int32)
    for it in range(KNN):
        m = jnp.max(x, axis=1, keepdims=True)
        p = jnp.min(jnp.where(x == m, lane, BIG), axis=1)[:, None]
        gi = jnp.max(jnp.where(lane == p, gidx, -1), axis=1)[:, None]
        accv = jnp.where(col == it, m, accv)
        acci = jnp.where(col == it, gi, acci)
        x = jnp.where(lane == p, NEG_INF, x)
    val_ref[...] = -accv
    idx_ref[...] = acci


def _final(cand, seg_ids):
    return pl.pallas_call(
        _final_body,
        grid=(NQ // QB,),
        in_specs=[
            pl.BlockSpec((QB, KNN, GROW), lambda qi: (qi, 0, 0)),
            pl.BlockSpec((QB, KNN), lambda qi: (qi, 0)),
        ],
        out_specs=[
            pl.BlockSpec((QB, KNN), lambda qi: (qi, 0)),
            pl.BlockSpec((QB, KNN), lambda qi: (qi, 0)),
        ],
        out_shape=[
            jax.ShapeDtypeStruct((NQ, KNN), jnp.float32),
            jax.ShapeDtypeStruct((NQ, KNN), jnp.int32),
        ],
    )(cand, seg_ids)


NC = 2
NS = 16
NGATHER = NQ * KNN
CHUNK = 128


def _make_sc_gather():
    nw = NC * NS
    per_w = NGATHER // nw
    nchunk = per_w // CHUNK
    mesh = plsc.VectorSubcoreMesh(
        core_axis_name="c", subcore_axis_name="s", num_cores=NC, num_subcores=NS)

    @functools.partial(
        pl.kernel,
        mesh=mesh,
        out_type=jax.ShapeDtypeStruct((NGATHER, GROW), jnp.float32),
        scratch_types=[
            pltpu.VMEM((nchunk, CHUNK), jnp.int32),
            pltpu.VMEM((CHUNK, GROW), jnp.float32),
            pltpu.SemaphoreType.DMA,
        ],
    )
    def gather_k(s_hbm, idx_hbm, out_hbm, idx_v, rows_v, sem):
        wid = lax.axis_index("s") * NC + lax.axis_index("c")
        pltpu.sync_copy(idx_hbm.at[pl.ds(wid * nchunk, nchunk)], idx_v)
        base = wid * per_w

        def body(j, carry):
            pltpu.async_copy(s_hbm.at[idx_v.at[j]], rows_v, sem).wait()
            pltpu.sync_copy(rows_v, out_hbm.at[pl.ds(base + j * CHUNK, CHUNK)])
            return carry

        lax.fori_loop(0, nchunk, body, 0)

    return gather_k


def kernel(queries, keys, k):
    del k
    keys_p = jnp.pad(keys, ((0, KPAD - NK), (0, 0)), constant_values=1000.0)
    ksq = jnp.sum(keys_p * keys_p, axis=1)
    s, m = _scores(queries, keys_p, ksq.reshape(1, KPAD))
    seg_ids, flat_idx = _segtop(m)
    s_rows = s.reshape(NQ * NROW, GROW)
    cand = _make_sc_gather()(
        s_rows, flat_idx.reshape(NGATHER // CHUNK, CHUNK)).reshape(NQ, KNN, GROW)
    vals, idxs = _final(cand, seg_ids)
    return vals, idxs
